# baseline (speedup 1.0000x reference)
import functools

import jax
import jax.numpy as jnp
from jax import lax
from jax.experimental import pallas as pl
from jax.experimental.pallas import tpu as pltpu
from jax.experimental.pallas import tpu_sc as plsc

N_NODES = 10000
N_PAD = 10240
FEAT = 64
NSUB = 16
EB = 128
LANES = 16
ROWS_PER_TILE = N_PAD // 16
ZROWS = 128
NBUF = 6
K0 = 120
K1 = 42
KTOT = K0 + K1
KMAX = max(K0, K1)


def _lane_broadcast(vec, lane):
  idx = jnp.full((LANES, 1), lane, dtype=jnp.int32)
  dn = lax.GatherDimensionNumbers(
      offset_dims=(), collapsed_slice_dims=(0,), start_index_map=(0,))
  return lax.gather(vec, idx, dn, (1,),
                    mode=lax.GatherScatterMode.PROMISE_IN_BOUNDS)


def _make_hop():
  mesh = plsc.VectorSubcoreMesh(core_axis_name="c", subcore_axis_name="s")
  assert K0 % NBUF == 0 and K1 % NBUF == 0

  @functools.partial(
      pl.kernel,
      out_type=jax.ShapeDtypeStruct((2, N_PAD, FEAT), jnp.float32),
      mesh=mesh,
      compiler_params=pltpu.CompilerParams(use_tc_tiling_on_sc=False),
      scratch_types=[
          pltpu.VMEM((KMAX, EB), jnp.int32),
          pltpu.VMEM((KMAX, EB), jnp.float32),
          pltpu.VMEM((NBUF, EB), jnp.int32),
          pltpu.VMEM((NBUF, EB), jnp.int32),
          [pltpu.VMEM((EB, FEAT), jnp.float32) for _ in range(NBUF)],
          pltpu.VMEM_SHARED((N_PAD, FEAT), jnp.float32),
          [pltpu.SemaphoreType.DMA for _ in range(NBUF)],
          [pltpu.SemaphoreType.DMA for _ in range(NBUF)],
      ],
  )
  def hop(table, packed, val, out, pk_v, val_v, src_v, dst_v, rows,
          acc, sg, ss):
    c = lax.axis_index("c")
    s = lax.axis_index("s")

    zv = jnp.zeros((LANES,), jnp.float32)

    @pl.loop(0, ZROWS)
    def _(r):
      for q in range(FEAT // LANES):
        rows[0][r, pl.ds(q * LANES, LANES)] = zv

    for t in range(ROWS_PER_TILE // ZROWS):
      pltpu.sync_copy(rows[0],
                      acc.at[pl.ds(s * ROWS_PER_TILE + t * ZROWS, ZROWS)])

    @pl.when(c == 0)
    def _():
      pltpu.sync_copy(packed.at[s, pl.ds(0, K0)], pk_v.at[pl.ds(0, K0)])
      pltpu.sync_copy(val.at[s, pl.ds(0, K0)], val_v.at[pl.ds(0, K0)])

    @pl.when(c == 1)
    def _():
      pltpu.sync_copy(packed.at[s, pl.ds(K0, K1)], pk_v.at[pl.ds(0, K1)])
      pltpu.sync_copy(val.at[s, pl.ds(K0, K1)], val_v.at[pl.ds(0, K1)])

    plsc.subcore_barrier()

    ngroups = jnp.where(c == 0, K0 // NBUF, K1 // NBUF)

    @pl.loop(0, ngroups)
    def _(j0):
      jbase = j0 * NBUF
      for b in range(NBUF):
        for q in range(EB // LANES):
          pk = pk_v[jbase + b, pl.ds(q * LANES, LANES)]
          src_v[b, pl.ds(q * LANES, LANES)] = lax.shift_right_logical(
              pk, jnp.full((LANES,), 14, jnp.int32))
      gd = [pltpu.async_copy(table.at[src_v.at[b]], rows[b], sg[b])
            for b in range(NBUF)]
      for b in range(NBUF):
        for q in range(EB // LANES):
          pk = pk_v[jbase + b, pl.ds(q * LANES, LANES)]
          dst_v[b, pl.ds(q * LANES, LANES)] = lax.bitwise_and(
              pk, jnp.full((LANES,), 16383, jnp.int32))
      sd = []
      for b in range(NBUF):
        gd[b].wait()

        @pl.loop(0, EB // LANES)
        def _(g):
          vals = val_v[jbase + b, pl.ds(g * LANES, LANES)]
          scales = [_lane_broadcast(vals, e) for e in range(LANES)]
          base = g * LANES
          for e4 in range(0, LANES, 4):
            loads = [(e4 + i, q,
                      rows[b][base + e4 + i, pl.ds(q * LANES, LANES)])
                     for i in range(4) for q in range(FEAT // LANES)]
            for e, q, rr in loads:
              rows[b][base + e, pl.ds(q * LANES, LANES)] = rr * scales[e]

        sd.append(pltpu.async_copy(rows[b], acc.at[dst_v.at[b]], ss[b],
                                   add=True))
      for d in sd:
        d.wait()

    plsc.subcore_barrier()

    for t in range(ROWS_PER_TILE // ZROWS):
      base = s * ROWS_PER_TILE + t * ZROWS
      pltpu.sync_copy(acc.at[pl.ds(base, ZROWS)], out.at[c, pl.ds(base, ZROWS)])

  return hop


def _matmul_body(x_ref, wt_ref, o_ref):
  o_ref[...] = jnp.dot(x_ref[...], wt_ref[...],
                       preferred_element_type=jnp.float32)


def _tc_matmul(x, wt):
  return pl.pallas_call(
      _matmul_body,
      out_shape=jax.ShapeDtypeStruct((x.shape[0], wt.shape[1]), jnp.float32),
  )(x, wt)


def _combine_body(p_ref, b_ref, o_ref):
  o_ref[...] = p_ref[0] + p_ref[1] + b_ref[...]


def _tc_combine(p, bias_row):
  return pl.pallas_call(
      _combine_body,
      out_shape=jax.ShapeDtypeStruct(p.shape[1:], jnp.float32),
  )(p, bias_row)


def kernel(x, adj_indices, adj_values, W, b):
  dst = adj_indices[0].astype(jnp.int32)
  src = adj_indices[1].astype(jnp.int32)
  val = adj_values.astype(jnp.float32)

  e = src.shape[0]
  e_pad = NSUB * KTOT * EB
  assert e_pad >= e
  pad = e_pad - e
  if pad:
    src = jnp.pad(src, (0, pad))
    dst = jnp.pad(dst, (0, pad))
    val = jnp.pad(val, (0, pad))
  packed = jnp.left_shift(src, 14) | dst
  packed_r = packed.reshape(NSUB, KTOT, EB)
  val_r = val.reshape(NSUB, KTOT, EB)

  z = _tc_matmul(x, W.T)
  z = jnp.pad(z, ((0, N_PAD - N_NODES), (0, 0)))
  hop = _make_hop()
  zero_row = jnp.zeros((1, FEAT), jnp.float32)
  p1 = hop(z, packed_r, val_r)
  h1 = _tc_combine(p1, zero_row)
  p2 = hop(h1, packed_r, val_r)
  out = _tc_combine(p2, b.reshape(1, FEAT).astype(jnp.float32))
  return out[:N_NODES]


# speedup vs baseline: 1.3051x; 1.3051x over previous
import functools

import jax
import jax.numpy as jnp
from jax import lax
from jax.experimental import pallas as pl
from jax.experimental.pallas import tpu as pltpu
from jax.experimental.pallas import tpu_sc as plsc

N_NODES = 10000
N_PAD = 10240
FEAT = 64
NSUB = 16
EB = 128
LANES = 16
ROWS_PER_TILE = N_PAD // 16
ZROWS = 128
NBUF = 4
K0 = 120
K1 = 40
KTOT = K0 + K1
KMAX = max(K0, K1)


def _lane_broadcast(vec, lane):
  idx = jnp.full((LANES, 1), lane, dtype=jnp.int32)
  dn = lax.GatherDimensionNumbers(
      offset_dims=(), collapsed_slice_dims=(0,), start_index_map=(0,))
  return lax.gather(vec, idx, dn, (1,),
                    mode=lax.GatherScatterMode.PROMISE_IN_BOUNDS)


def _make_hop():
  mesh = plsc.VectorSubcoreMesh(core_axis_name="c", subcore_axis_name="s")
  assert K0 % NBUF == 0 and K1 % NBUF == 0

  @functools.partial(
      pl.kernel,
      out_type=jax.ShapeDtypeStruct((2, N_PAD, FEAT), jnp.float32),
      mesh=mesh,
      compiler_params=pltpu.CompilerParams(use_tc_tiling_on_sc=False),
      scratch_types=[
          pltpu.VMEM((KMAX, EB), jnp.int32),
          pltpu.VMEM((KMAX, EB), jnp.float32),
          pltpu.VMEM((NBUF, EB), jnp.int32),
          pltpu.VMEM((NBUF, EB), jnp.int32),
          [pltpu.VMEM((EB, FEAT), jnp.float32) for _ in range(NBUF)],
          pltpu.VMEM_SHARED((N_PAD, FEAT), jnp.float32),
          [pltpu.SemaphoreType.DMA for _ in range(NBUF)],
          [pltpu.SemaphoreType.DMA for _ in range(NBUF)],
      ],
  )
  def hop(table, packed, val, out, pk_v, val_v, src_v, dst_v, rows,
          acc, sg, ss):
    c = lax.axis_index("c")
    s = lax.axis_index("s")

    zv = jnp.zeros((LANES,), jnp.float32)

    @pl.loop(0, ZROWS)
    def _(r):
      for q in range(FEAT // LANES):
        rows[0][r, pl.ds(q * LANES, LANES)] = zv

    for t in range(ROWS_PER_TILE // ZROWS):
      pltpu.sync_copy(rows[0],
                      acc.at[pl.ds(s * ROWS_PER_TILE + t * ZROWS, ZROWS)])

    @pl.when(c == 0)
    def _():
      pltpu.sync_copy(packed.at[s, pl.ds(0, K0)], pk_v.at[pl.ds(0, K0)])
      pltpu.sync_copy(val.at[s, pl.ds(0, K0)], val_v.at[pl.ds(0, K0)])

    @pl.when(c == 1)
    def _():
      pltpu.sync_copy(packed.at[s, pl.ds(K0, K1)], pk_v.at[pl.ds(0, K1)])
      pltpu.sync_copy(val.at[s, pl.ds(K0, K1)], val_v.at[pl.ds(0, K1)])

    plsc.subcore_barrier()

    ngroups = jnp.where(c == 0, K0 // NBUF, K1 // NBUF)

    @pl.loop(0, ngroups)
    def _(j0):
      jbase = j0 * NBUF
      for b in range(NBUF):
        for q in range(EB // LANES):
          pk = pk_v[jbase + b, pl.ds(q * LANES, LANES)]
          src_v[b, pl.ds(q * LANES, LANES)] = lax.shift_right_logical(
              pk, jnp.full((LANES,), 14, jnp.int32))
      gd = [pltpu.async_copy(table.at[src_v.at[b]], rows[b], sg[b])
            for b in range(NBUF)]
      for b in range(NBUF):
        for q in range(EB // LANES):
          pk = pk_v[jbase + b, pl.ds(q * LANES, LANES)]
          dst_v[b, pl.ds(q * LANES, LANES)] = lax.bitwise_and(
              pk, jnp.full((LANES,), 16383, jnp.int32))
      sd = []
      for b in range(NBUF):
        gd[b].wait()

        @pl.loop(0, EB // LANES)
        def _(g):
          vals = val_v[jbase + b, pl.ds(g * LANES, LANES)]
          scales = [_lane_broadcast(vals, e) for e in range(LANES)]
          base = g * LANES
          for e4 in range(0, LANES, 4):
            loads = [(e4 + i, q,
                      rows[b][base + e4 + i, pl.ds(q * LANES, LANES)])
                     for i in range(4) for q in range(FEAT // LANES)]
            for e, q, rr in loads:
              rows[b][base + e, pl.ds(q * LANES, LANES)] = rr * scales[e]

        sd.append(pltpu.async_copy(rows[b], acc.at[dst_v.at[b]], ss[b],
                                   add=True))
      for d in sd:
        d.wait()

    plsc.subcore_barrier()

    for t in range(ROWS_PER_TILE // ZROWS):
      base = s * ROWS_PER_TILE + t * ZROWS
      pltpu.sync_copy(acc.at[pl.ds(base, ZROWS)], out.at[c, pl.ds(base, ZROWS)])

  return hop


def _matmul_body(x_ref, wt_ref, o_ref):
  o_ref[...] = jnp.dot(x_ref[...], wt_ref[...],
                       preferred_element_type=jnp.float32)


def _tc_matmul(x, wt):
  return pl.pallas_call(
      _matmul_body,
      out_shape=jax.ShapeDtypeStruct((x.shape[0], wt.shape[1]), jnp.float32),
  )(x, wt)


def _combine_body(p_ref, b_ref, o_ref):
  o_ref[...] = p_ref[0] + p_ref[1] + b_ref[...]


def _tc_combine(p, bias_row):
  return pl.pallas_call(
      _combine_body,
      out_shape=jax.ShapeDtypeStruct(p.shape[1:], jnp.float32),
  )(p, bias_row)


def kernel(x, adj_indices, adj_values, W, b):
  dst = adj_indices[0].astype(jnp.int32)
  src = adj_indices[1].astype(jnp.int32)
  val = adj_values.astype(jnp.float32)

  e = src.shape[0]
  e_pad = NSUB * KTOT * EB
  assert e_pad >= e
  pad = e_pad - e
  if pad:
    src = jnp.pad(src, (0, pad))
    dst = jnp.pad(dst, (0, pad))
    val = jnp.pad(val, (0, pad))
  packed = jnp.left_shift(src, 14) | dst
  packed_r = packed.reshape(NSUB, KTOT, EB)
  val_r = val.reshape(NSUB, KTOT, EB)

  z = _tc_matmul(x, W.T)
  z = jnp.pad(z, ((0, N_PAD - N_NODES), (0, 0)))
  hop = _make_hop()
  zero_row = jnp.zeros((1, FEAT), jnp.float32)
  p1 = hop(z, packed_r, val_r)
  h1 = _tc_combine(p1, zero_row)
  p2 = hop(h1, packed_r, val_r)
  out = _tc_combine(p2, b.reshape(1, FEAT).astype(jnp.float32))
  return out[:N_NODES]


# speedup vs baseline: 1.3429x; 1.0289x over previous
import functools

import jax
import jax.numpy as jnp
from jax import lax
from jax.experimental import pallas as pl
from jax.experimental.pallas import tpu as pltpu
from jax.experimental.pallas import tpu_sc as plsc

N_NODES = 10000
N_PAD = 10240
FEAT = 64
NSUB = 16
EB = 128
LANES = 16
ROWS_PER_TILE = N_PAD // 16
ZROWS = 128
NBUF = 5
K0 = 120
K1 = 40
KTOT = K0 + K1
KMAX = max(K0, K1)


def _lane_broadcast(vec, lane):
  idx = jnp.full((LANES, 1), lane, dtype=jnp.int32)
  dn = lax.GatherDimensionNumbers(
      offset_dims=(), collapsed_slice_dims=(0,), start_index_map=(0,))
  return lax.gather(vec, idx, dn, (1,),
                    mode=lax.GatherScatterMode.PROMISE_IN_BOUNDS)


def _make_hop():
  mesh = plsc.VectorSubcoreMesh(core_axis_name="c", subcore_axis_name="s")
  assert K0 % NBUF == 0 and K1 % NBUF == 0

  @functools.partial(
      pl.kernel,
      out_type=jax.ShapeDtypeStruct((2, N_PAD, FEAT), jnp.float32),
      mesh=mesh,
      compiler_params=pltpu.CompilerParams(use_tc_tiling_on_sc=False),
      scratch_types=[
          pltpu.VMEM((KMAX, EB), jnp.int32),
          pltpu.VMEM((KMAX, EB), jnp.float32),
          pltpu.VMEM((NBUF, EB), jnp.int32),
          pltpu.VMEM((NBUF, EB), jnp.int32),
          [pltpu.VMEM((EB, FEAT), jnp.float32) for _ in range(NBUF)],
          pltpu.VMEM_SHARED((N_PAD, FEAT), jnp.float32),
          [pltpu.SemaphoreType.DMA for _ in range(NBUF)],
          [pltpu.SemaphoreType.DMA for _ in range(NBUF)],
      ],
  )
  def hop(table, packed, val, out, pk_v, val_v, src_v, dst_v, rows,
          acc, sg, ss):
    c = lax.axis_index("c")
    s = lax.axis_index("s")

    zv = jnp.zeros((LANES,), jnp.float32)

    @pl.loop(0, ZROWS)
    def _(r):
      for q in range(FEAT // LANES):
        rows[0][r, pl.ds(q * LANES, LANES)] = zv

    for t in range(ROWS_PER_TILE // ZROWS):
      pltpu.sync_copy(rows[0],
                      acc.at[pl.ds(s * ROWS_PER_TILE + t * ZROWS, ZROWS)])

    @pl.when(c == 0)
    def _():
      pltpu.sync_copy(packed.at[s, pl.ds(0, K0)], pk_v.at[pl.ds(0, K0)])
      pltpu.sync_copy(val.at[s, pl.ds(0, K0)], val_v.at[pl.ds(0, K0)])

    @pl.when(c == 1)
    def _():
      pltpu.sync_copy(packed.at[s, pl.ds(K0, K1)], pk_v.at[pl.ds(0, K1)])
      pltpu.sync_copy(val.at[s, pl.ds(K0, K1)], val_v.at[pl.ds(0, K1)])

    plsc.subcore_barrier()

    ngroups = jnp.where(c == 0, K0 // NBUF, K1 // NBUF)

    @pl.loop(0, ngroups)
    def _(j0):
      jbase = j0 * NBUF
      for b in range(NBUF):
        for q in range(EB // LANES):
          pk = pk_v[jbase + b, pl.ds(q * LANES, LANES)]
          src_v[b, pl.ds(q * LANES, LANES)] = lax.shift_right_logical(
              pk, jnp.full((LANES,), 14, jnp.int32))
      gd = [pltpu.async_copy(table.at[src_v.at[b]], rows[b], sg[b])
            for b in range(NBUF)]
      for b in range(NBUF):
        for q in range(EB // LANES):
          pk = pk_v[jbase + b, pl.ds(q * LANES, LANES)]
          dst_v[b, pl.ds(q * LANES, LANES)] = lax.bitwise_and(
              pk, jnp.full((LANES,), 16383, jnp.int32))
      sd = []
      for b in range(NBUF):
        gd[b].wait()

        @pl.loop(0, EB // LANES)
        def _(g):
          vals = val_v[jbase + b, pl.ds(g * LANES, LANES)]
          scales = [_lane_broadcast(vals, e) for e in range(LANES)]
          base = g * LANES
          for e4 in range(0, LANES, 4):
            loads = [(e4 + i, q,
                      rows[b][base + e4 + i, pl.ds(q * LANES, LANES)])
                     for i in range(4) for q in range(FEAT // LANES)]
            for e, q, rr in loads:
              rows[b][base + e, pl.ds(q * LANES, LANES)] = rr * scales[e]

        sd.append(pltpu.async_copy(rows[b], acc.at[dst_v.at[b]], ss[b],
                                   add=True))
      for d in sd:
        d.wait()

    plsc.subcore_barrier()

    for t in range(ROWS_PER_TILE // ZROWS):
      base = s * ROWS_PER_TILE + t * ZROWS
      pltpu.sync_copy(acc.at[pl.ds(base, ZROWS)], out.at[c, pl.ds(base, ZROWS)])

  return hop


def _matmul_body(x_ref, wt_ref, o_ref):
  o_ref[...] = jnp.dot(x_ref[...], wt_ref[...],
                       preferred_element_type=jnp.float32)


def _tc_matmul(x, wt):
  return pl.pallas_call(
      _matmul_body,
      out_shape=jax.ShapeDtypeStruct((x.shape[0], wt.shape[1]), jnp.float32),
  )(x, wt)


def _combine_body(p_ref, b_ref, o_ref):
  o_ref[...] = p_ref[0] + p_ref[1] + b_ref[...]


def _tc_combine(p, bias_row):
  return pl.pallas_call(
      _combine_body,
      out_shape=jax.ShapeDtypeStruct(p.shape[1:], jnp.float32),
  )(p, bias_row)


def kernel(x, adj_indices, adj_values, W, b):
  dst = adj_indices[0].astype(jnp.int32)
  src = adj_indices[1].astype(jnp.int32)
  val = adj_values.astype(jnp.float32)

  e = src.shape[0]
  e_pad = NSUB * KTOT * EB
  assert e_pad >= e
  pad = e_pad - e
  if pad:
    src = jnp.pad(src, (0, pad))
    dst = jnp.pad(dst, (0, pad))
    val = jnp.pad(val, (0, pad))
  packed = jnp.left_shift(src, 14) | dst
  packed_r = packed.reshape(NSUB, KTOT, EB)
  val_r = val.reshape(NSUB, KTOT, EB)

  z = _tc_matmul(x, W.T)
  z = jnp.pad(z, ((0, N_PAD - N_NODES), (0, 0)))
  hop = _make_hop()
  zero_row = jnp.zeros((1, FEAT), jnp.float32)
  p1 = hop(z, packed_r, val_r)
  h1 = _tc_combine(p1, zero_row)
  p2 = hop(h1, packed_r, val_r)
  out = _tc_combine(p2, b.reshape(1, FEAT).astype(jnp.float32))
  return out[:N_NODES]
